# baseline (speedup 1.0000x reference)
import numpy as np
import jax
import jax.numpy as jnp
from jax.experimental import pallas as pl
from jax.experimental.pallas import tpu as pltpu

S = 128
K = 64
RB = 256

_TINY = np.float32(np.finfo(np.float32).tiny)
_ROT_A = (13, 15, 26, 6)
_ROT_B = (17, 29, 16, 24)


def _np_threefry2x32(k0, k1, x0, x1):
    k0 = np.uint32(k0); k1 = np.uint32(k1)
    ks = (k0, k1, np.uint32(k0 ^ k1 ^ np.uint32(0x1BD11BDA)))
    x0 = np.uint32(np.uint64(x0) + ks[0]); x1 = np.uint32(np.uint64(x1) + ks[1])
    rots = (_ROT_A, _ROT_B, _ROT_A, _ROT_B, _ROT_A)
    inj = ((1, 2), (2, 0), (0, 1), (1, 2), (2, 0))
    for i in range(5):
        for r in rots[i]:
            x0 = np.uint32((np.uint64(x0) + np.uint64(x1)) & np.uint64(0xFFFFFFFF))
            x1 = np.uint32(((np.uint64(x1) << np.uint64(r)) | (np.uint64(x1) >> np.uint64(32 - r))) & np.uint64(0xFFFFFFFF))
            x1 = np.uint32(x1 ^ x0)
        a, b = inj[i]
        x0 = np.uint32((np.uint64(x0) + np.uint64(ks[a])) & np.uint64(0xFFFFFFFF))
        x1 = np.uint32((np.uint64(x1) + np.uint64(ks[b]) + np.uint64(i + 1)) & np.uint64(0xFFFFFFFF))
    return x0, x1


_KEY0, _KEY1 = _np_threefry2x32(0, 1, 0, 0)
_KEY2 = np.uint32(_KEY0 ^ _KEY1 ^ np.uint32(0x1BD11BDA))


def _threefry_bits_from_x1(x1):
    ks = (np.uint32(_KEY0), np.uint32(_KEY1), _KEY2)
    rots = (_ROT_A, _ROT_B, _ROT_A, _ROT_B, _ROT_A)
    inj = ((1, 2), (2, 0), (0, 1), (1, 2), (2, 0))
    x0 = ks[0]
    for i in range(5):
        for r in rots[i]:
            x0 = x0 + x1
            x1 = (x1 << np.uint32(r)) | (x1 >> np.uint32(32 - r))
            x1 = x1 ^ x0
        a, b = inj[i]
        x0 = x0 + ks[a]
        x1 = x1 + np.uint32(np.uint64(ks[b]) + np.uint64(i + 1) & np.uint64(0xFFFFFFFF))
    return x0 ^ x1


def _resample_kernel(w_ref, td2_ref, sd2_ref, p3_ref,
                     inds_ref, otd_ref, osd_ref, opt_ref, iota_ref):
    blk = pl.program_id(0)
    base = blk * (RB * S * K)

    @pl.when(blk == 0)
    def _init():
        ri = jax.lax.broadcasted_iota(jnp.int32, (RB, K, S), 0)
        kk = jax.lax.broadcasted_iota(jnp.int32, (RB, K, S), 1)
        ss = jax.lax.broadcasted_iota(jnp.int32, (RB, K, S), 2)
        iota_ref[...] = (ri * (S * K) + ss * K + kk
                         + jnp.int32(np.int32(np.uint32(_KEY1).view(np.int32))))

    x1_init = (iota_ref[...] + base).astype(jnp.uint32)
    bits = _threefry_bits_from_x1(x1_init)
    ssf = jax.lax.broadcasted_iota(jnp.int32, (RB, K, S), 2).astype(jnp.float32)
    fl = jax.lax.bitcast_convert_type(
        (bits >> np.uint32(9)) | np.uint32(0x3F800000), jnp.float32) - np.float32(1.0)
    u = jnp.maximum(fl, _TINY)
    logw = jnp.log(jnp.maximum(w_ref[...], _TINY))
    vals = logw[:, None, :] - jnp.log(-jnp.log(u))
    m = jnp.max(vals, axis=-1, keepdims=True)
    indf = jnp.min(jnp.where(vals == m, ssf, np.float32(S)),
                   axis=-1)
    inds_ref[...] = indf.astype(jnp.int32)
    ohf = jnp.where(ssf == indf[:, :, None], np.float32(1.0), np.float32(0.0))

    def dot_nt(a, b):
        return jax.lax.dot_general(
            a, b, dimension_numbers=(((1,), (1,)), ((), ())),
            precision=jax.lax.Precision.HIGHEST,
            preferred_element_type=jnp.float32)

    a7 = jnp.concatenate([td2_ref[...], sd2_ref[...], p3_ref[...]], axis=1)
    for r in range(RB):
        g7 = dot_nt(a7[r], ohf[r])
        otd_ref[r] = g7[0:2]
        osd_ref[r] = g7[2:4]
        opt_ref[r] = g7[4:7]


def kernel(weights, points, tdist, sdist, num_resample):
    R = weights.shape[0]
    del num_resample
    td2 = jnp.stack([tdist[:, :S], tdist[:, 1:S + 1]], axis=1)
    sd2 = jnp.stack([sdist[:, :S], sdist[:, 1:S + 1]], axis=1)
    p3 = jnp.transpose(points, (0, 2, 1))

    outs = pl.pallas_call(
        _resample_kernel,
        grid=(R // RB,),
        in_specs=[pl.BlockSpec((RB, S), lambda i: (i, 0)),
                  pl.BlockSpec((RB, 2, S), lambda i: (i, 0, 0)),
                  pl.BlockSpec((RB, 2, S), lambda i: (i, 0, 0)),
                  pl.BlockSpec((RB, 3, S), lambda i: (i, 0, 0))],
        scratch_shapes=[pltpu.VMEM((RB, K, S), jnp.int32)],
        out_specs=[pl.BlockSpec((RB, K), lambda i: (i, 0)),
                   pl.BlockSpec((RB, 2, K), lambda i: (i, 0, 0)),
                   pl.BlockSpec((RB, 2, K), lambda i: (i, 0, 0)),
                   pl.BlockSpec((RB, 3, K), lambda i: (i, 0, 0))],
        out_shape=[jax.ShapeDtypeStruct((R, K), jnp.int32),
                   jax.ShapeDtypeStruct((R, 2, K), jnp.float32),
                   jax.ShapeDtypeStruct((R, 2, K), jnp.float32),
                   jax.ShapeDtypeStruct((R, 3, K), jnp.float32)],
        compiler_params=pltpu.CompilerParams(
            dimension_semantics=("parallel",)),
    )(weights, td2, sd2, p3)
    inds, otd, osd, opt = outs

    f_weights = jnp.ones((R, K), jnp.float32)
    f_tdist = otd.reshape(R, 2 * K)
    f_sdist = osd.reshape(R, 2 * K)
    f_points = jnp.swapaxes(opt, 1, 2)
    return (f_weights, f_points, f_tdist, f_sdist, inds)


# speedup vs baseline: 1.2368x; 1.2368x over previous
import numpy as np
import jax
import jax.numpy as jnp
from jax.experimental import pallas as pl
from jax.experimental.pallas import tpu as pltpu

S = 128
K = 64
RB = 128

_TINY = np.float32(np.finfo(np.float32).tiny)
_ROT_A = (13, 15, 26, 6)
_ROT_B = (17, 29, 16, 24)


def _np_threefry2x32(k0, k1, x0, x1):
    k0 = np.uint32(k0); k1 = np.uint32(k1)
    ks = (k0, k1, np.uint32(k0 ^ k1 ^ np.uint32(0x1BD11BDA)))
    x0 = np.uint32(np.uint64(x0) + ks[0]); x1 = np.uint32(np.uint64(x1) + ks[1])
    rots = (_ROT_A, _ROT_B, _ROT_A, _ROT_B, _ROT_A)
    inj = ((1, 2), (2, 0), (0, 1), (1, 2), (2, 0))
    for i in range(5):
        for r in rots[i]:
            x0 = np.uint32((np.uint64(x0) + np.uint64(x1)) & np.uint64(0xFFFFFFFF))
            x1 = np.uint32(((np.uint64(x1) << np.uint64(r)) | (np.uint64(x1) >> np.uint64(32 - r))) & np.uint64(0xFFFFFFFF))
            x1 = np.uint32(x1 ^ x0)
        a, b = inj[i]
        x0 = np.uint32((np.uint64(x0) + np.uint64(ks[a])) & np.uint64(0xFFFFFFFF))
        x1 = np.uint32((np.uint64(x1) + np.uint64(ks[b]) + np.uint64(i + 1)) & np.uint64(0xFFFFFFFF))
    return x0, x1


_KEY0, _KEY1 = _np_threefry2x32(0, 1, 0, 0)
_KEY2 = np.uint32(_KEY0 ^ _KEY1 ^ np.uint32(0x1BD11BDA))


def _threefry_bits_from_x1(x1):
    ks = (np.uint32(_KEY0), np.uint32(_KEY1), _KEY2)
    rots = (_ROT_A, _ROT_B, _ROT_A, _ROT_B, _ROT_A)
    inj = ((1, 2), (2, 0), (0, 1), (1, 2), (2, 0))
    x0 = ks[0]
    for i in range(5):
        for r in rots[i]:
            x0 = x0 + x1
            x1 = (x1 << np.uint32(r)) | (x1 >> np.uint32(32 - r))
            x1 = x1 ^ x0
        a, b = inj[i]
        x0 = x0 + ks[a]
        x1 = x1 + np.uint32(np.uint64(ks[b]) + np.uint64(i + 1) & np.uint64(0xFFFFFFFF))
    return x0 ^ x1


def _resample_kernel(w_ref, td2_ref, sd2_ref, p3_ref,
                     inds_ref, otd_ref, osd_ref, opt_ref, iota_ref):
    blk = pl.program_id(0)
    base = blk * (RB * S * K)

    @pl.when(blk == 0)
    def _init():
        ri = jax.lax.broadcasted_iota(jnp.int32, (RB, K, S), 0)
        kk = jax.lax.broadcasted_iota(jnp.int32, (RB, K, S), 1)
        ss = jax.lax.broadcasted_iota(jnp.int32, (RB, K, S), 2)
        iota_ref[...] = (ri * (S * K) + ss * K + kk
                         + jnp.int32(np.int32(np.uint32(_KEY1).view(np.int32))))

    x1_init = (iota_ref[...] + base).astype(jnp.uint32)
    bits = _threefry_bits_from_x1(x1_init)
    ssf = jax.lax.broadcasted_iota(jnp.int32, (RB, K, S), 2).astype(jnp.float32)
    fl = jax.lax.bitcast_convert_type(
        (bits >> np.uint32(9)) | np.uint32(0x3F800000), jnp.float32) - np.float32(1.0)
    u = jnp.maximum(fl, _TINY)
    logw = jnp.log(jnp.maximum(w_ref[...], _TINY))
    vals = logw[:, None, :] - jnp.log(-jnp.log(u))
    m = jnp.max(vals, axis=-1, keepdims=True)
    indf = jnp.min(jnp.where(vals == m, ssf, np.float32(S)),
                   axis=-1)
    inds_ref[...] = indf.astype(jnp.int32)
    ohf = jnp.where(ssf == indf[:, :, None], np.float32(1.0), np.float32(0.0))

    def dot_nt(a, b):
        return jax.lax.dot_general(
            a, b, dimension_numbers=(((1,), (1,)), ((), ())),
            precision=jax.lax.Precision.HIGHEST,
            preferred_element_type=jnp.float32)

    a7 = jnp.concatenate([td2_ref[...], sd2_ref[...], p3_ref[...]], axis=1)
    for r in range(RB):
        g7 = dot_nt(a7[r], ohf[r])
        otd_ref[r] = g7[0:2]
        osd_ref[r] = g7[2:4]
        opt_ref[r] = g7[4:7]


def kernel(weights, points, tdist, sdist, num_resample):
    R = weights.shape[0]
    del num_resample
    td2 = jnp.stack([tdist[:, :S], tdist[:, 1:S + 1]], axis=1)
    sd2 = jnp.stack([sdist[:, :S], sdist[:, 1:S + 1]], axis=1)
    p3 = jnp.transpose(points, (0, 2, 1))

    outs = pl.pallas_call(
        _resample_kernel,
        grid=(R // RB,),
        in_specs=[pl.BlockSpec((RB, S), lambda i: (i, 0)),
                  pl.BlockSpec((RB, 2, S), lambda i: (i, 0, 0)),
                  pl.BlockSpec((RB, 2, S), lambda i: (i, 0, 0)),
                  pl.BlockSpec((RB, 3, S), lambda i: (i, 0, 0))],
        scratch_shapes=[pltpu.VMEM((RB, K, S), jnp.int32)],
        out_specs=[pl.BlockSpec((RB, K), lambda i: (i, 0)),
                   pl.BlockSpec((RB, 2, K), lambda i: (i, 0, 0)),
                   pl.BlockSpec((RB, 2, K), lambda i: (i, 0, 0)),
                   pl.BlockSpec((RB, 3, K), lambda i: (i, 0, 0))],
        out_shape=[jax.ShapeDtypeStruct((R, K), jnp.int32),
                   jax.ShapeDtypeStruct((R, 2, K), jnp.float32),
                   jax.ShapeDtypeStruct((R, 2, K), jnp.float32),
                   jax.ShapeDtypeStruct((R, 3, K), jnp.float32)],
        compiler_params=pltpu.CompilerParams(
            dimension_semantics=("arbitrary",)),
    )(weights, td2, sd2, p3)
    inds, otd, osd, opt = outs

    f_weights = jnp.ones((R, K), jnp.float32)
    f_tdist = otd.reshape(R, 2 * K)
    f_sdist = osd.reshape(R, 2 * K)
    f_points = jnp.swapaxes(opt, 1, 2)
    return (f_weights, f_points, f_tdist, f_sdist, inds)
